# initial kernel scaffold (unmeasured)
import jax
import jax.numpy as jnp
from jax import lax
from jax.experimental import pallas as pl
from jax.experimental.pallas import tpu as pltpu

N_DEV = 4


def kernel(x, w_mat):
    m_per, k = x.shape
    _, n_per = w_mat.shape
    half = m_per // 2
    m_glob = N_DEV * m_per

    x_bf = x.astype(jnp.bfloat16)
    w_bf = w_mat.astype(jnp.bfloat16)

    def body(x_ref, w_ref, out_ref, cw, ccw, w_vmem, y_stage,
             amax_tile, amax_gather, cw_send, cw_recv, ccw_send, ccw_recv,
             load_sems, y_sems, ax_send_sems, ax_recv_sems):
        my = lax.axis_index("i")
        left = lax.rem(my + N_DEV - 1, N_DEV)
        right = lax.rem(my + 1, N_DEV)

        w_copy = pltpu.make_async_copy(w_ref, w_vmem, load_sems.at[0])
        xa_copy = pltpu.make_async_copy(
            x_ref.at[pl.ds(0, half)], cw.at[0], load_sems.at[1])
        xb_copy = pltpu.make_async_copy(
            x_ref.at[pl.ds(half, half)], ccw.at[0], load_sems.at[2])
        w_copy.start()
        xa_copy.start()
        xb_copy.start()

        bsem = pltpu.get_barrier_semaphore()
        for nbr in (left, right):
            pl.semaphore_signal(
                bsem, inc=1, device_id=(nbr,),
                device_id_type=pl.DeviceIdType.MESH)
        pl.semaphore_wait(bsem, 2)

        xa_copy.wait()
        xb_copy.wait()

        def hop(buf, ssems, rsems, h, tgt):
            return pltpu.make_async_remote_copy(
                src_ref=buf.at[h], dst_ref=buf.at[h + 1],
                send_sem=ssems.at[h], recv_sem=rsems.at[h],
                device_id=(tgt,), device_id_type=pl.DeviceIdType.MESH)

        cw_rdma = [hop(cw, cw_send, cw_recv, h, right)
                   for h in range(N_DEV - 1)]
        ccw_rdma = [hop(ccw, ccw_send, ccw_recv, h, left)
                    for h in range(N_DEV - 1)]
        cw_rdma[0].start()
        ccw_rdma[0].start()

        w_copy.wait()

        state = {"amax": jnp.float32(0.0), "dma": [None, None], "slot": 0}

        def emit(xhalf, row_start):
            slot = state["slot"]
            if state["dma"][slot] is not None:
                state["dma"][slot].wait()
            yv = jnp.dot(xhalf, w_vmem[...],
                         preferred_element_type=jnp.float32)
            state["amax"] = jnp.maximum(state["amax"],
                                        jnp.max(jnp.abs(yv)))
            y_stage[slot] = yv
            dma = pltpu.make_async_copy(
                y_stage.at[slot], out_ref.at[pl.ds(row_start, half)],
                y_sems.at[slot])
            dma.start()
            state["dma"][slot] = dma
            state["slot"] = 1 - slot

        emit(cw[0], my * m_per)
        emit(ccw[0], my * m_per + half)

        for h in range(N_DEV - 1):
            cw_rdma[h].wait_recv()
            if h + 1 < N_DEV - 1:
                cw_rdma[h + 1].start()
            ccw_rdma[h].wait_recv()
            if h + 1 < N_DEV - 1:
                ccw_rdma[h + 1].start()
            o_cw = lax.rem(my + N_DEV - h - 1, N_DEV)
            emit(cw[h + 1], o_cw * m_per)
            o_ccw = lax.rem(my + h + 1, N_DEV)
            emit(ccw[h + 1], o_ccw * m_per + half)

        for r in cw_rdma + ccw_rdma:
            r.wait_send()
        for d in state["dma"]:
            if d is not None:
                d.wait()

        amax_tile[...] = jnp.full((8, 128), state["amax"], jnp.float32)
        ax_sends = []
        for off in (1, 2, 3):
            tgt = lax.rem(my + off, N_DEV)
            s = pltpu.make_async_remote_copy(
                src_ref=amax_tile, dst_ref=amax_gather.at[my],
                send_sem=ax_send_sems.at[off - 1],
                recv_sem=ax_recv_sems.at[my],
                device_id=(tgt,), device_id_type=pl.DeviceIdType.MESH)
            s.start()
            ax_sends.append(s)

        gmax = state["amax"]
        for off in (1, 2, 3):
            src = lax.rem(my + off, N_DEV)
            r = pltpu.make_async_remote_copy(
                src_ref=amax_tile, dst_ref=amax_gather.at[src],
                send_sem=ax_send_sems.at[0],
                recv_sem=ax_recv_sems.at[src],
                device_id=(left,), device_id_type=pl.DeviceIdType.MESH)
            r.wait_recv()
            gmax = jnp.maximum(gmax, jnp.max(amax_gather[pl.ds(src, 1)]))
        for s in ax_sends:
            s.wait_send()

        scale = gmax / 127.0
        n_tiles = m_glob // half
        dmas_in = [None] * n_tiles
        dmas_out = [None] * n_tiles

        def start_in(t):
            slot = t % 2
            d = pltpu.make_async_copy(
                out_ref.at[pl.ds(t * half, half)], y_stage.at[slot],
                y_sems.at[slot])
            d.start()
            dmas_in[t] = d

        start_in(0)
        for t in range(n_tiles):
            slot = t % 2
            if t + 1 < n_tiles:
                if t >= 1:
                    dmas_out[t - 1].wait()
                start_in(t + 1)
            dmas_in[t].wait()
            v = y_stage[slot]
            q = jnp.clip(jnp.round(v / scale), -127.0, 127.0) * scale
            y_stage[slot] = q
            d = pltpu.make_async_copy(
                y_stage.at[slot], out_ref.at[pl.ds(t * half, half)],
                y_sems.at[slot])
            d.start()
            dmas_out[t] = d
        dmas_out[n_tiles - 2].wait()
        dmas_out[n_tiles - 1].wait()

    return pl.pallas_call(
        body,
        out_shape=jax.ShapeDtypeStruct((m_glob, n_per), jnp.float32),
        in_specs=[pl.BlockSpec(memory_space=pltpu.ANY),
                  pl.BlockSpec(memory_space=pltpu.ANY)],
        out_specs=pl.BlockSpec(memory_space=pltpu.ANY),
        scratch_shapes=[
            pltpu.VMEM((N_DEV, half, k), jnp.bfloat16),
            pltpu.VMEM((N_DEV, half, k), jnp.bfloat16),
            pltpu.VMEM((k, n_per), jnp.bfloat16),
            pltpu.VMEM((2, half, n_per), jnp.float32),
            pltpu.VMEM((8, 128), jnp.float32),
            pltpu.VMEM((N_DEV, 8, 128), jnp.float32),
            pltpu.SemaphoreType.DMA((N_DEV - 1,)),
            pltpu.SemaphoreType.DMA((N_DEV - 1,)),
            pltpu.SemaphoreType.DMA((N_DEV - 1,)),
            pltpu.SemaphoreType.DMA((N_DEV - 1,)),
            pltpu.SemaphoreType.DMA((3,)),
            pltpu.SemaphoreType.DMA((2,)),
            pltpu.SemaphoreType.DMA((3,)),
            pltpu.SemaphoreType.DMA((N_DEV,)),
        ],
        compiler_params=pltpu.CompilerParams(
            collective_id=0,
            vmem_limit_bytes=64 * 1024 * 1024,
        ),
    )(x_bf, w_bf)


# baseline (device time: 250435 ns/iter reference)
import jax
import jax.numpy as jnp
from jax import lax
from jax.experimental import pallas as pl
from jax.experimental.pallas import tpu as pltpu

N_DEV = 4


def kernel(x, w_mat):
    m_per, k = x.shape
    _, n_per = w_mat.shape
    half = m_per // 2
    m_glob = N_DEV * m_per

    x_bf = x.astype(jnp.bfloat16)
    w_bf = w_mat.astype(jnp.bfloat16)

    def body(x_ref, w_ref, out_ref, cw, ccw, w_vmem, y_stage,
             amax_tile, amax_gather, cw_send, cw_recv, ccw_send, ccw_recv,
             load_sems, y_sems, ax_send_sems, ax_recv_sems):
        my = lax.axis_index("i")
        left = lax.rem(my + N_DEV - 1, N_DEV)
        right = lax.rem(my + 1, N_DEV)

        w_copy = pltpu.make_async_copy(w_ref, w_vmem, load_sems.at[0])
        xa_copy = pltpu.make_async_copy(
            x_ref.at[pl.ds(0, half)], cw.at[0], load_sems.at[1])
        xb_copy = pltpu.make_async_copy(
            x_ref.at[pl.ds(half, half)], ccw.at[0], load_sems.at[2])
        w_copy.start()
        xa_copy.start()
        xb_copy.start()

        bsem = pltpu.get_barrier_semaphore()
        for nbr in (left, right):
            pl.semaphore_signal(
                bsem, inc=1, device_id=(nbr,),
                device_id_type=pl.DeviceIdType.MESH)
        pl.semaphore_wait(bsem, 2)

        xa_copy.wait()
        xb_copy.wait()

        def hop(buf, ssems, rsems, h, tgt):
            return pltpu.make_async_remote_copy(
                src_ref=buf.at[h], dst_ref=buf.at[h + 1],
                send_sem=ssems.at[h], recv_sem=rsems.at[h],
                device_id=(tgt,), device_id_type=pl.DeviceIdType.MESH)

        cw_rdma = [hop(cw, cw_send, cw_recv, h, right)
                   for h in range(N_DEV - 1)]
        ccw_rdma = [hop(ccw, ccw_send, ccw_recv, h, left)
                    for h in range(N_DEV - 1)]
        cw_rdma[0].start()
        ccw_rdma[0].start()

        w_copy.wait()

        state = {"amax": jnp.float32(0.0), "dma": [None, None], "slot": 0}

        def emit(xhalf, row_start):
            slot = state["slot"]
            if state["dma"][slot] is not None:
                state["dma"][slot].wait()
            yv = jnp.dot(xhalf, w_vmem[...],
                         preferred_element_type=jnp.float32)
            state["amax"] = jnp.maximum(state["amax"],
                                        jnp.max(jnp.abs(yv)))
            y_stage[slot] = yv
            dma = pltpu.make_async_copy(
                y_stage.at[slot], out_ref.at[pl.ds(row_start, half)],
                y_sems.at[slot])
            dma.start()
            state["dma"][slot] = dma
            state["slot"] = 1 - slot

        emit(cw[0], my * m_per)
        emit(ccw[0], my * m_per + half)

        for h in range(N_DEV - 1):
            cw_rdma[h].wait_recv()
            if h + 1 < N_DEV - 1:
                cw_rdma[h + 1].start()
            ccw_rdma[h].wait_recv()
            if h + 1 < N_DEV - 1:
                ccw_rdma[h + 1].start()
            o_cw = lax.rem(my + N_DEV - h - 1, N_DEV)
            emit(cw[h + 1], o_cw * m_per)
            o_ccw = lax.rem(my + h + 1, N_DEV)
            emit(ccw[h + 1], o_ccw * m_per + half)

        for r in cw_rdma + ccw_rdma:
            r.wait_send()
        for d in state["dma"]:
            if d is not None:
                d.wait()

        amax_tile[...] = jnp.full((8, 128), state["amax"], jnp.float32)
        ax_sends = []
        for off in (1, 2, 3):
            tgt = lax.rem(my + off, N_DEV)
            s = pltpu.make_async_remote_copy(
                src_ref=amax_tile, dst_ref=amax_gather.at[my],
                send_sem=ax_send_sems.at[off - 1],
                recv_sem=ax_recv_sems.at[my],
                device_id=(tgt,), device_id_type=pl.DeviceIdType.MESH)
            s.start()
            ax_sends.append(s)

        gmax = state["amax"]
        for off in (1, 2, 3):
            src = lax.rem(my + off, N_DEV)
            r = pltpu.make_async_remote_copy(
                src_ref=amax_tile, dst_ref=amax_gather.at[src],
                send_sem=ax_send_sems.at[0],
                recv_sem=ax_recv_sems.at[src],
                device_id=(left,), device_id_type=pl.DeviceIdType.MESH)
            r.wait_recv()
            gmax = jnp.maximum(gmax, jnp.max(amax_gather[pl.ds(src, 1)]))
        for s in ax_sends:
            s.wait_send()

        scale = gmax / 127.0
        n_tiles = m_glob // half
        dmas_in = [None] * n_tiles
        dmas_out = [None] * n_tiles

        def start_in(t):
            slot = t % 2
            d = pltpu.make_async_copy(
                out_ref.at[pl.ds(t * half, half)], y_stage.at[slot],
                y_sems.at[slot])
            d.start()
            dmas_in[t] = d

        start_in(0)
        for t in range(n_tiles):
            slot = t % 2
            if t + 1 < n_tiles:
                if t >= 1:
                    dmas_out[t - 1].wait()
                start_in(t + 1)
            dmas_in[t].wait()
            v = y_stage[slot]
            q = jnp.clip(jnp.round(v / scale), -127.0, 127.0) * scale
            y_stage[slot] = q
            d = pltpu.make_async_copy(
                y_stage.at[slot], out_ref.at[pl.ds(t * half, half)],
                y_sems.at[slot])
            d.start()
            dmas_out[t] = d
        dmas_out[n_tiles - 2].wait()
        dmas_out[n_tiles - 1].wait()

    return pl.pallas_call(
        body,
        out_shape=jax.ShapeDtypeStruct((m_glob, n_per), jnp.float32),
        in_specs=[pl.BlockSpec(memory_space=pl.ANY),
                  pl.BlockSpec(memory_space=pl.ANY)],
        out_specs=pl.BlockSpec(memory_space=pl.ANY),
        scratch_shapes=[
            pltpu.VMEM((N_DEV, half, k), jnp.bfloat16),
            pltpu.VMEM((N_DEV, half, k), jnp.bfloat16),
            pltpu.VMEM((k, n_per), jnp.bfloat16),
            pltpu.VMEM((2, half, n_per), jnp.float32),
            pltpu.VMEM((8, 128), jnp.float32),
            pltpu.VMEM((N_DEV, 8, 128), jnp.float32),
            pltpu.SemaphoreType.DMA((N_DEV - 1,)),
            pltpu.SemaphoreType.DMA((N_DEV - 1,)),
            pltpu.SemaphoreType.DMA((N_DEV - 1,)),
            pltpu.SemaphoreType.DMA((N_DEV - 1,)),
            pltpu.SemaphoreType.DMA((3,)),
            pltpu.SemaphoreType.DMA((2,)),
            pltpu.SemaphoreType.DMA((3,)),
            pltpu.SemaphoreType.DMA((N_DEV,)),
        ],
        compiler_params=pltpu.CompilerParams(
            collective_id=0,
            vmem_limit_bytes=64 * 1024 * 1024,
        ),
    )(x_bf, w_bf)


# device time: 232674 ns/iter; 1.0763x vs baseline; 1.0763x over previous
import jax
import jax.numpy as jnp
from jax import lax
from jax.experimental import pallas as pl
from jax.experimental.pallas import tpu as pltpu

N_DEV = 4


def kernel(x, w_mat):
    m_per, k = x.shape
    _, n_per = w_mat.shape
    half = m_per // 2
    m_glob = N_DEV * m_per
    xt = 64
    wt = 256

    def body(x_ref, w_ref, out_ref, cw, ccw, w_vmem, y_stage,
             xstage, wstage, amax_tile, amax_gather,
             cw_send, cw_recv, ccw_send, ccw_recv,
             load_sems, y_sems, ax_send_sems, ax_recv_sems):
        my = lax.axis_index("i")
        left = lax.rem(my + N_DEV - 1, N_DEV)
        right = lax.rem(my + 1, N_DEV)

        bsem = pltpu.get_barrier_semaphore()
        for nbr in (left, right):
            pl.semaphore_signal(
                bsem, inc=1, device_id=(nbr,),
                device_id_type=pl.DeviceIdType.MESH)

        def xcast(dstbuf, base):
            def bdy(i, c):
                d = pltpu.make_async_copy(
                    x_ref.at[pl.ds(base + i * xt, xt)], xstage,
                    load_sems.at[0])
                d.start()
                d.wait()
                dstbuf[0, pl.ds(i * xt, xt)] = (
                    xstage[...].astype(jnp.bfloat16))
                return c
            lax.fori_loop(0, half // xt, bdy, 0)

        xcast(cw, 0)
        xcast(ccw, half)

        pl.semaphore_wait(bsem, 2)

        def hop(buf, ssems, rsems, h, tgt):
            return pltpu.make_async_remote_copy(
                src_ref=buf.at[h], dst_ref=buf.at[h + 1],
                send_sem=ssems.at[h], recv_sem=rsems.at[h],
                device_id=(tgt,), device_id_type=pl.DeviceIdType.MESH)

        cw_rdma = [hop(cw, cw_send, cw_recv, h, right)
                   for h in range(N_DEV - 1)]
        ccw_rdma = [hop(ccw, ccw_send, ccw_recv, h, left)
                    for h in range(N_DEV - 1)]
        cw_rdma[0].start()
        ccw_rdma[0].start()

        def wbdy(i, c):
            d = pltpu.make_async_copy(
                w_ref.at[pl.ds(i * wt, wt)], wstage, load_sems.at[1])
            d.start()
            d.wait()
            w_vmem[pl.ds(i * wt, wt)] = wstage[...].astype(jnp.bfloat16)
            return c
        lax.fori_loop(0, k // wt, wbdy, 0)

        state = {"amax": jnp.float32(0.0), "dma": [None, None], "slot": 0}

        def emit(xhalf, row_start):
            slot = state["slot"]
            if state["dma"][slot] is not None:
                state["dma"][slot].wait()
            yv = jnp.dot(xhalf, w_vmem[...],
                         preferred_element_type=jnp.float32)
            state["amax"] = jnp.maximum(state["amax"],
                                        jnp.max(jnp.abs(yv)))
            y_stage[slot] = yv
            dma = pltpu.make_async_copy(
                y_stage.at[slot], out_ref.at[pl.ds(row_start, half)],
                y_sems.at[slot])
            dma.start()
            state["dma"][slot] = dma
            state["slot"] = 1 - slot

        emit(cw[0], my * m_per)
        emit(ccw[0], my * m_per + half)

        for h in range(N_DEV - 1):
            cw_rdma[h].wait_recv()
            if h + 1 < N_DEV - 1:
                cw_rdma[h + 1].start()
            ccw_rdma[h].wait_recv()
            if h + 1 < N_DEV - 1:
                ccw_rdma[h + 1].start()
            o_cw = lax.rem(my + N_DEV - h - 1, N_DEV)
            emit(cw[h + 1], o_cw * m_per)
            o_ccw = lax.rem(my + h + 1, N_DEV)
            emit(ccw[h + 1], o_ccw * m_per + half)

        for r in cw_rdma + ccw_rdma:
            r.wait_send()
        for d in state["dma"]:
            if d is not None:
                d.wait()

        amax_tile[...] = jnp.full((8, 128), state["amax"], jnp.float32)
        ax_sends = []
        for off in (1, 2, 3):
            tgt = lax.rem(my + off, N_DEV)
            s = pltpu.make_async_remote_copy(
                src_ref=amax_tile, dst_ref=amax_gather.at[my],
                send_sem=ax_send_sems.at[off - 1],
                recv_sem=ax_recv_sems.at[my],
                device_id=(tgt,), device_id_type=pl.DeviceIdType.MESH)
            s.start()
            ax_sends.append(s)

        n_tiles = m_glob // half
        dmas_in = [None] * n_tiles
        dmas_out = [None] * n_tiles

        def start_in(t):
            slot = t % 2
            d = pltpu.make_async_copy(
                out_ref.at[pl.ds(t * half, half)], y_stage.at[slot],
                y_sems.at[slot])
            d.start()
            dmas_in[t] = d

        start_in(0)
        start_in(1)

        gmax = state["amax"]
        for off in (1, 2, 3):
            src = lax.rem(my + off, N_DEV)
            r = pltpu.make_async_remote_copy(
                src_ref=amax_tile, dst_ref=amax_gather.at[src],
                send_sem=ax_send_sems.at[0],
                recv_sem=ax_recv_sems.at[src],
                device_id=(left,), device_id_type=pl.DeviceIdType.MESH)
            r.wait_recv()
            gmax = jnp.maximum(gmax, jnp.max(amax_gather[pl.ds(src, 1)]))
        for s in ax_sends:
            s.wait_send()

        scale = gmax / 127.0
        for t in range(n_tiles):
            slot = t % 2
            if 1 <= t and t + 1 < n_tiles:
                dmas_out[t - 1].wait()
                start_in(t + 1)
            dmas_in[t].wait()
            v = y_stage[slot]
            q = jnp.clip(jnp.round(v / scale), -127.0, 127.0) * scale
            y_stage[slot] = q
            d = pltpu.make_async_copy(
                y_stage.at[slot], out_ref.at[pl.ds(t * half, half)],
                y_sems.at[slot])
            d.start()
            dmas_out[t] = d
        dmas_out[n_tiles - 2].wait()
        dmas_out[n_tiles - 1].wait()

    return pl.pallas_call(
        body,
        out_shape=jax.ShapeDtypeStruct((m_glob, n_per), jnp.float32),
        in_specs=[pl.BlockSpec(memory_space=pl.ANY),
                  pl.BlockSpec(memory_space=pl.ANY)],
        out_specs=pl.BlockSpec(memory_space=pl.ANY),
        scratch_shapes=[
            pltpu.VMEM((N_DEV, half, k), jnp.bfloat16),
            pltpu.VMEM((N_DEV, half, k), jnp.bfloat16),
            pltpu.VMEM((k, n_per), jnp.bfloat16),
            pltpu.VMEM((2, half, n_per), jnp.float32),
            pltpu.VMEM((xt, k), jnp.float32),
            pltpu.VMEM((wt, n_per), jnp.float32),
            pltpu.VMEM((8, 128), jnp.float32),
            pltpu.VMEM((N_DEV, 8, 128), jnp.float32),
            pltpu.SemaphoreType.DMA((N_DEV - 1,)),
            pltpu.SemaphoreType.DMA((N_DEV - 1,)),
            pltpu.SemaphoreType.DMA((N_DEV - 1,)),
            pltpu.SemaphoreType.DMA((N_DEV - 1,)),
            pltpu.SemaphoreType.DMA((2,)),
            pltpu.SemaphoreType.DMA((2,)),
            pltpu.SemaphoreType.DMA((3,)),
            pltpu.SemaphoreType.DMA((N_DEV,)),
        ],
        compiler_params=pltpu.CompilerParams(
            collective_id=0,
            vmem_limit_bytes=64 * 1024 * 1024,
        ),
    )(x, w_mat)


# device time: 230802 ns/iter; 1.0851x vs baseline; 1.0081x over previous
import jax
import jax.numpy as jnp
from jax import lax
from jax.experimental import pallas as pl
from jax.experimental.pallas import tpu as pltpu

N_DEV = 4


def kernel(x, w_mat):
    m_per, k = x.shape
    _, n_per = w_mat.shape
    half = m_per // 2
    m_glob = N_DEV * m_per
    xt = 32
    wt = 128

    def body(x_ref, w_ref, out_ref, cw, ccw, w_vmem, y_stage,
             xstage, wstage, amax_tile, amax_gather,
             cw_send, cw_recv, ccw_send, ccw_recv,
             load_sems, y_sems, ax_send_sems, ax_recv_sems):
        my = lax.axis_index("i")
        left = lax.rem(my + N_DEV - 1, N_DEV)
        right = lax.rem(my + 1, N_DEV)

        bsem = pltpu.get_barrier_semaphore()
        for nbr in (left, right):
            pl.semaphore_signal(
                bsem, inc=1, device_id=(nbr,),
                device_id_type=pl.DeviceIdType.MESH)

        def xcast(dstbuf, base, n):
            def xdma(i):
                return pltpu.make_async_copy(
                    x_ref.at[pl.ds(base + i * xt, xt)],
                    xstage.at[lax.rem(i, 2)],
                    load_sems.at[lax.rem(i, 2)])

            xdma(0).start()

            def bdy(i, c):
                @pl.when(i + 1 < n)
                def _():
                    xdma(i + 1).start()
                xdma(i).wait()
                dstbuf[0, pl.ds(i * xt, xt)] = (
                    xstage[lax.rem(i, 2)].astype(jnp.bfloat16))
                return c
            lax.fori_loop(0, n, bdy, 0)

        xcast(cw, 0, half // xt)
        xcast(ccw, half, half // xt)

        pl.semaphore_wait(bsem, 2)

        def hop(buf, ssems, rsems, h, tgt):
            return pltpu.make_async_remote_copy(
                src_ref=buf.at[h], dst_ref=buf.at[h + 1],
                send_sem=ssems.at[h], recv_sem=rsems.at[h],
                device_id=(tgt,), device_id_type=pl.DeviceIdType.MESH)

        cw_rdma = [hop(cw, cw_send, cw_recv, h, right)
                   for h in range(N_DEV - 1)]
        ccw_rdma = [hop(ccw, ccw_send, ccw_recv, h, left)
                    for h in range(N_DEV - 1)]
        cw_rdma[0].start()
        ccw_rdma[0].start()

        def wdma(i):
            return pltpu.make_async_copy(
                w_ref.at[pl.ds(i * wt, wt)], wstage.at[lax.rem(i, 2)],
                load_sems.at[2 + lax.rem(i, 2)])

        wdma(0).start()

        def wbdy(i, c):
            @pl.when(i + 1 < k // wt)
            def _():
                wdma(i + 1).start()
            wdma(i).wait()
            w_vmem[pl.ds(i * wt, wt)] = (
                wstage[lax.rem(i, 2)].astype(jnp.bfloat16))
            return c
        lax.fori_loop(0, k // wt, wbdy, 0)

        state = {"amax": jnp.float32(0.0), "dma": [None, None], "slot": 0}

        def emit(xhalf, row_start):
            slot = state["slot"]
            if state["dma"][slot] is not None:
                state["dma"][slot].wait()
            yv = jnp.dot(xhalf, w_vmem[...],
                         preferred_element_type=jnp.float32)
            state["amax"] = jnp.maximum(state["amax"],
                                        jnp.max(jnp.abs(yv)))
            y_stage[slot] = yv
            dma = pltpu.make_async_copy(
                y_stage.at[slot], out_ref.at[pl.ds(row_start, half)],
                y_sems.at[slot])
            dma.start()
            state["dma"][slot] = dma
            state["slot"] = 1 - slot

        emit(cw[0], my * m_per)
        emit(ccw[0], my * m_per + half)

        for h in range(N_DEV - 1):
            cw_rdma[h].wait_recv()
            if h + 1 < N_DEV - 1:
                cw_rdma[h + 1].start()
            ccw_rdma[h].wait_recv()
            if h + 1 < N_DEV - 1:
                ccw_rdma[h + 1].start()
            o_cw = lax.rem(my + N_DEV - h - 1, N_DEV)
            emit(cw[h + 1], o_cw * m_per)
            o_ccw = lax.rem(my + h + 1, N_DEV)
            emit(ccw[h + 1], o_ccw * m_per + half)

        for r in cw_rdma + ccw_rdma:
            r.wait_send()
        for d in state["dma"]:
            if d is not None:
                d.wait()

        amax_tile[...] = jnp.full((8, 128), state["amax"], jnp.float32)
        ax_sends = []
        for off in (1, 2, 3):
            tgt = lax.rem(my + off, N_DEV)
            s = pltpu.make_async_remote_copy(
                src_ref=amax_tile, dst_ref=amax_gather.at[my],
                send_sem=ax_send_sems.at[off - 1],
                recv_sem=ax_recv_sems.at[my],
                device_id=(tgt,), device_id_type=pl.DeviceIdType.MESH)
            s.start()
            ax_sends.append(s)

        n_tiles = m_glob // half
        dmas_in = [None] * n_tiles
        dmas_out = [None] * n_tiles

        def start_in(t):
            slot = t % 2
            d = pltpu.make_async_copy(
                out_ref.at[pl.ds(t * half, half)], y_stage.at[slot],
                y_sems.at[slot])
            d.start()
            dmas_in[t] = d

        start_in(0)
        start_in(1)

        gmax = state["amax"]
        for off in (1, 2, 3):
            src = lax.rem(my + off, N_DEV)
            r = pltpu.make_async_remote_copy(
                src_ref=amax_tile, dst_ref=amax_gather.at[src],
                send_sem=ax_send_sems.at[0],
                recv_sem=ax_recv_sems.at[src],
                device_id=(left,), device_id_type=pl.DeviceIdType.MESH)
            r.wait_recv()
            gmax = jnp.maximum(gmax, jnp.max(amax_gather[pl.ds(src, 1)]))
        for s in ax_sends:
            s.wait_send()

        scale = gmax / 127.0
        for t in range(n_tiles):
            slot = t % 2
            if 1 <= t and t + 1 < n_tiles:
                dmas_out[t - 1].wait()
                start_in(t + 1)
            dmas_in[t].wait()
            v = y_stage[slot]
            q = jnp.clip(jnp.round(v / scale), -127.0, 127.0) * scale
            y_stage[slot] = q
            d = pltpu.make_async_copy(
                y_stage.at[slot], out_ref.at[pl.ds(t * half, half)],
                y_sems.at[slot])
            d.start()
            dmas_out[t] = d
        dmas_out[n_tiles - 2].wait()
        dmas_out[n_tiles - 1].wait()

    return pl.pallas_call(
        body,
        out_shape=jax.ShapeDtypeStruct((m_glob, n_per), jnp.float32),
        in_specs=[pl.BlockSpec(memory_space=pl.ANY),
                  pl.BlockSpec(memory_space=pl.ANY)],
        out_specs=pl.BlockSpec(memory_space=pl.ANY),
        scratch_shapes=[
            pltpu.VMEM((N_DEV, half, k), jnp.bfloat16),
            pltpu.VMEM((N_DEV, half, k), jnp.bfloat16),
            pltpu.VMEM((k, n_per), jnp.bfloat16),
            pltpu.VMEM((2, half, n_per), jnp.float32),
            pltpu.VMEM((2, xt, k), jnp.float32),
            pltpu.VMEM((2, wt, n_per), jnp.float32),
            pltpu.VMEM((8, 128), jnp.float32),
            pltpu.VMEM((N_DEV, 8, 128), jnp.float32),
            pltpu.SemaphoreType.DMA((N_DEV - 1,)),
            pltpu.SemaphoreType.DMA((N_DEV - 1,)),
            pltpu.SemaphoreType.DMA((N_DEV - 1,)),
            pltpu.SemaphoreType.DMA((N_DEV - 1,)),
            pltpu.SemaphoreType.DMA((4,)),
            pltpu.SemaphoreType.DMA((2,)),
            pltpu.SemaphoreType.DMA((3,)),
            pltpu.SemaphoreType.DMA((N_DEV,)),
        ],
        compiler_params=pltpu.CompilerParams(
            collective_id=0,
            vmem_limit_bytes=64 * 1024 * 1024,
        ),
    )(x, w_mat)


# device time: 220334 ns/iter; 1.1366x vs baseline; 1.0475x over previous
import jax
import jax.numpy as jnp
from jax import lax
from jax.experimental import pallas as pl
from jax.experimental.pallas import tpu as pltpu

N_DEV = 4


def kernel(x, w_mat):
    m_per, k = x.shape
    _, n_per = w_mat.shape
    half = m_per // 2
    m_glob = N_DEV * m_per
    xt = 32
    wt = 128

    def body(x_ref, w_ref, out_ref, cw, ccw, w_vmem, y_stage,
             xstage, wstage, amax_tile, amax_gather,
             cw_send, cw_recv, ccw_send, ccw_recv,
             load_sems, y_sems, ax_send_sems, ax_recv_sems):
        my = lax.axis_index("i")
        left = lax.rem(my + N_DEV - 1, N_DEV)
        right = lax.rem(my + 1, N_DEV)

        bsem = pltpu.get_barrier_semaphore()
        for nbr in (left, right):
            pl.semaphore_signal(
                bsem, inc=1, device_id=(nbr,),
                device_id_type=pl.DeviceIdType.MESH)

        def xcast(dstbuf, base, n):
            def xdma(i):
                return pltpu.make_async_copy(
                    x_ref.at[pl.ds(base + i * xt, xt)],
                    xstage.at[lax.rem(i, 2)],
                    load_sems.at[lax.rem(i, 2)])

            xdma(0).start()

            def bdy(i, c):
                @pl.when(i + 1 < n)
                def _():
                    xdma(i + 1).start()
                xdma(i).wait()
                dstbuf[0, pl.ds(i * xt, xt)] = (
                    xstage[lax.rem(i, 2)].astype(jnp.bfloat16))
                return c
            lax.fori_loop(0, n, bdy, 0)

        xcast(cw, 0, half // xt)
        xcast(ccw, half, half // xt)

        pl.semaphore_wait(bsem, 2)

        def hop(buf, ssems, rsems, h, tgt):
            return pltpu.make_async_remote_copy(
                src_ref=buf.at[h], dst_ref=buf.at[h + 1],
                send_sem=ssems.at[h], recv_sem=rsems.at[h],
                device_id=(tgt,), device_id_type=pl.DeviceIdType.MESH)

        quarter = half // 2

        def hop2(buf, ssems, rsems, j, tgt):
            return pltpu.make_async_remote_copy(
                src_ref=buf.at[2, pl.ds(j * quarter, quarter)],
                dst_ref=buf.at[3, pl.ds(j * quarter, quarter)],
                send_sem=ssems.at[2 + j], recv_sem=rsems.at[2 + j],
                device_id=(tgt,), device_id_type=pl.DeviceIdType.MESH)

        cw_rdma = [hop(cw, cw_send, cw_recv, h, right) for h in range(2)]
        ccw_rdma = [hop(ccw, ccw_send, ccw_recv, h, left) for h in range(2)]
        cw2 = [hop2(cw, cw_send, cw_recv, j, right) for j in range(2)]
        ccw2 = [hop2(ccw, ccw_send, ccw_recv, j, left) for j in range(2)]
        cw_rdma[0].start()
        ccw_rdma[0].start()

        def wdma(i):
            return pltpu.make_async_copy(
                w_ref.at[pl.ds(i * wt, wt)], wstage.at[lax.rem(i, 2)],
                load_sems.at[2 + lax.rem(i, 2)])

        wdma(0).start()

        def wbdy(i, c):
            @pl.when(i + 1 < k // wt)
            def _():
                wdma(i + 1).start()
            wdma(i).wait()
            w_vmem[pl.ds(i * wt, wt)] = (
                wstage[lax.rem(i, 2)].astype(jnp.bfloat16))
            return c
        lax.fori_loop(0, k // wt, wbdy, 0)

        state = {"amax": jnp.float32(0.0), "dma": [None, None], "slot": 0}

        def emit(xpart, row_start, rows=half):
            slot = state["slot"]
            if state["dma"][slot] is not None:
                state["dma"][slot].wait()
            yv = jnp.dot(xpart, w_vmem[...],
                         preferred_element_type=jnp.float32)
            state["amax"] = jnp.maximum(state["amax"],
                                        jnp.max(jnp.abs(yv)))
            y_stage[slot, pl.ds(0, rows)] = yv
            dma = pltpu.make_async_copy(
                y_stage.at[slot, pl.ds(0, rows)],
                out_ref.at[pl.ds(row_start, rows)],
                y_sems.at[slot])
            dma.start()
            state["dma"][slot] = dma
            state["slot"] = 1 - slot

        emit(cw[0], my * m_per)
        emit(ccw[0], my * m_per + half)

        for h in range(2):
            cw_rdma[h].wait_recv()
            if h == 0:
                cw_rdma[1].start()
            else:
                cw2[0].start()
                cw2[1].start()
            ccw_rdma[h].wait_recv()
            if h == 0:
                ccw_rdma[1].start()
            else:
                ccw2[0].start()
                ccw2[1].start()
            o_cw = lax.rem(my + N_DEV - h - 1, N_DEV)
            emit(cw[h + 1], o_cw * m_per)
            o_ccw = lax.rem(my + h + 1, N_DEV)
            emit(ccw[h + 1], o_ccw * m_per + half)

        o_cw = lax.rem(my + 1, N_DEV)
        o_ccw = lax.rem(my + 3, N_DEV)
        for j in range(2):
            cw2[j].wait_recv()
            emit(cw[3, pl.ds(j * quarter, quarter)],
                 o_cw * m_per + j * quarter, rows=quarter)
            ccw2[j].wait_recv()
            emit(ccw[3, pl.ds(j * quarter, quarter)],
                 o_ccw * m_per + half + j * quarter, rows=quarter)

        for r in cw_rdma + ccw_rdma + cw2 + ccw2:
            r.wait_send()
        for d in state["dma"]:
            if d is not None:
                d.wait()

        amax_tile[...] = jnp.full((8, 128), state["amax"], jnp.float32)
        ax_sends = []
        for off in (1, 2, 3):
            tgt = lax.rem(my + off, N_DEV)
            s = pltpu.make_async_remote_copy(
                src_ref=amax_tile, dst_ref=amax_gather.at[my],
                send_sem=ax_send_sems.at[off - 1],
                recv_sem=ax_recv_sems.at[my],
                device_id=(tgt,), device_id_type=pl.DeviceIdType.MESH)
            s.start()
            ax_sends.append(s)

        n_tiles = m_glob // half
        dmas_in = [None] * n_tiles
        dmas_out = [None] * n_tiles

        def start_in(t):
            slot = t % 2
            d = pltpu.make_async_copy(
                out_ref.at[pl.ds(t * half, half)], y_stage.at[slot],
                y_sems.at[slot])
            d.start()
            dmas_in[t] = d

        start_in(0)
        start_in(1)

        gmax = state["amax"]
        for off in (1, 2, 3):
            src = lax.rem(my + off, N_DEV)
            r = pltpu.make_async_remote_copy(
                src_ref=amax_tile, dst_ref=amax_gather.at[src],
                send_sem=ax_send_sems.at[0],
                recv_sem=ax_recv_sems.at[src],
                device_id=(left,), device_id_type=pl.DeviceIdType.MESH)
            r.wait_recv()
            gmax = jnp.maximum(gmax, jnp.max(amax_gather[pl.ds(src, 1)]))
        for s in ax_sends:
            s.wait_send()

        scale = gmax / 127.0
        for t in range(n_tiles):
            slot = t % 2
            if 1 <= t and t + 1 < n_tiles:
                dmas_out[t - 1].wait()
                start_in(t + 1)
            dmas_in[t].wait()
            v = y_stage[slot]
            q = jnp.clip(jnp.round(v / scale), -127.0, 127.0) * scale
            y_stage[slot] = q
            d = pltpu.make_async_copy(
                y_stage.at[slot], out_ref.at[pl.ds(t * half, half)],
                y_sems.at[slot])
            d.start()
            dmas_out[t] = d
        dmas_out[n_tiles - 2].wait()
        dmas_out[n_tiles - 1].wait()

    return pl.pallas_call(
        body,
        out_shape=jax.ShapeDtypeStruct((m_glob, n_per), jnp.float32),
        in_specs=[pl.BlockSpec(memory_space=pl.ANY),
                  pl.BlockSpec(memory_space=pl.ANY)],
        out_specs=pl.BlockSpec(memory_space=pl.ANY),
        scratch_shapes=[
            pltpu.VMEM((N_DEV, half, k), jnp.bfloat16),
            pltpu.VMEM((N_DEV, half, k), jnp.bfloat16),
            pltpu.VMEM((k, n_per), jnp.bfloat16),
            pltpu.VMEM((2, half, n_per), jnp.float32),
            pltpu.VMEM((2, xt, k), jnp.float32),
            pltpu.VMEM((2, wt, n_per), jnp.float32),
            pltpu.VMEM((8, 128), jnp.float32),
            pltpu.VMEM((N_DEV, 8, 128), jnp.float32),
            pltpu.SemaphoreType.DMA((4,)),
            pltpu.SemaphoreType.DMA((4,)),
            pltpu.SemaphoreType.DMA((4,)),
            pltpu.SemaphoreType.DMA((4,)),
            pltpu.SemaphoreType.DMA((4,)),
            pltpu.SemaphoreType.DMA((2,)),
            pltpu.SemaphoreType.DMA((3,)),
            pltpu.SemaphoreType.DMA((N_DEV,)),
        ],
        compiler_params=pltpu.CompilerParams(
            collective_id=0,
            vmem_limit_bytes=64 * 1024 * 1024,
        ),
    )(x, w_mat)


# device time: 213650 ns/iter; 1.1722x vs baseline; 1.0313x over previous
import jax
import jax.numpy as jnp
from jax import lax
from jax.experimental import pallas as pl
from jax.experimental.pallas import tpu as pltpu

N_DEV = 4


def kernel(x, w_mat):
    m_per, k = x.shape
    _, n_per = w_mat.shape
    half = m_per // 2
    m_glob = N_DEV * m_per
    xt = 32
    wt = 128

    def body(x_ref, w_ref, out_ref, cw, ccw, w_vmem, y_stage,
             xstage, wstage, amax_tile, amax_gather,
             cw_send, cw_recv, ccw_send, ccw_recv,
             load_sems, y_sems, ax_send_sems, ax_recv_sems):
        my = lax.axis_index("i")
        left = lax.rem(my + N_DEV - 1, N_DEV)
        right = lax.rem(my + 1, N_DEV)

        bsem = pltpu.get_barrier_semaphore()
        for nbr in (left, right):
            pl.semaphore_signal(
                bsem, inc=1, device_id=(nbr,),
                device_id_type=pl.DeviceIdType.MESH)

        quarter = half // 2

        def sub(buf, ssems, rsems, h, j, tgt):
            return pltpu.make_async_remote_copy(
                src_ref=buf.at[h, pl.ds(j * quarter, quarter)],
                dst_ref=buf.at[h + 1, pl.ds(j * quarter, quarter)],
                send_sem=ssems.at[2 * h + j], recv_sem=rsems.at[2 * h + j],
                device_id=(tgt,), device_id_type=pl.DeviceIdType.MESH)

        cwS = [[sub(cw, cw_send, cw_recv, h, j, right) for j in range(2)]
               for h in range(N_DEV - 1)]
        ccwS = [[sub(ccw, ccw_send, ccw_recv, h, j, left) for j in range(2)]
                for h in range(N_DEV - 1)]

        def xcast(dstbuf, dstrow, srcrow, n):
            def xdma(i):
                return pltpu.make_async_copy(
                    x_ref.at[pl.ds(srcrow + i * xt, xt)],
                    xstage.at[lax.rem(i, 2)],
                    load_sems.at[lax.rem(i, 2)])

            xdma(0).start()

            def bdy(i, c):
                @pl.when(i + 1 < n)
                def _():
                    xdma(i + 1).start()
                xdma(i).wait()
                dstbuf[0, pl.ds(dstrow + i * xt, xt)] = (
                    xstage[lax.rem(i, 2)].astype(jnp.bfloat16))
                return c
            lax.fori_loop(0, n, bdy, 0)

        nq = quarter // xt
        xcast(cw, 0, 0, nq)
        pl.semaphore_wait(bsem, 2)
        cwS[0][0].start()
        xcast(ccw, 0, half, nq)
        ccwS[0][0].start()
        xcast(cw, quarter, quarter, nq)
        cwS[0][1].start()
        xcast(ccw, quarter, half + quarter, nq)
        ccwS[0][1].start()

        def wdma(i):
            return pltpu.make_async_copy(
                w_ref.at[pl.ds(i * wt, wt)], wstage.at[lax.rem(i, 2)],
                load_sems.at[2 + lax.rem(i, 2)])

        wdma(0).start()

        def wbdy(i, c):
            @pl.when(i + 1 < k // wt)
            def _():
                wdma(i + 1).start()
            wdma(i).wait()
            w_vmem[pl.ds(i * wt, wt)] = (
                wstage[lax.rem(i, 2)].astype(jnp.bfloat16))
            return c
        lax.fori_loop(0, k // wt, wbdy, 0)

        state = {"amax": jnp.float32(0.0), "dma": [None, None], "slot": 0}

        def emit(xpart, row_start, rows=half):
            slot = state["slot"]
            if state["dma"][slot] is not None:
                state["dma"][slot].wait()
            yv = jnp.dot(xpart, w_vmem[...],
                         preferred_element_type=jnp.float32)
            state["amax"] = jnp.maximum(state["amax"],
                                        jnp.max(jnp.abs(yv)))
            y_stage[slot, pl.ds(0, rows)] = yv
            dma = pltpu.make_async_copy(
                y_stage.at[slot, pl.ds(0, rows)],
                out_ref.at[pl.ds(row_start, rows)],
                y_sems.at[slot])
            dma.start()
            state["dma"][slot] = dma
            state["slot"] = 1 - slot

        emit(cw[0], my * m_per)
        emit(ccw[0], my * m_per + half)

        for h in range(N_DEV - 1):
            o_cw = lax.rem(my + N_DEV - h - 1, N_DEV)
            o_ccw = lax.rem(my + h + 1, N_DEV)
            for j in range(2):
                cwS[h][j].wait_recv()
                if h + 1 < N_DEV - 1:
                    cwS[h + 1][j].start()
                ccwS[h][j].wait_recv()
                if h + 1 < N_DEV - 1:
                    ccwS[h + 1][j].start()
                emit(cw[h + 1, pl.ds(j * quarter, quarter)],
                     o_cw * m_per + j * quarter, rows=quarter)
                emit(ccw[h + 1, pl.ds(j * quarter, quarter)],
                     o_ccw * m_per + half + j * quarter, rows=quarter)

        amax_tile[...] = jnp.full((8, 128), state["amax"], jnp.float32)
        ax_sends = []
        for off in (1, 2, 3):
            tgt = lax.rem(my + off, N_DEV)
            s = pltpu.make_async_remote_copy(
                src_ref=amax_tile, dst_ref=amax_gather.at[my],
                send_sem=ax_send_sems.at[off - 1],
                recv_sem=ax_recv_sems.at[my],
                device_id=(tgt,), device_id_type=pl.DeviceIdType.MESH)
            s.start()
            ax_sends.append(s)

        for r in cwS + ccwS:
            r[0].wait_send()
            r[1].wait_send()
        for d in state["dma"]:
            if d is not None:
                d.wait()

        n_tiles = m_glob // half
        dmas_in = [None] * n_tiles
        dmas_out = [None] * n_tiles

        def start_in(t):
            slot = t % 2
            d = pltpu.make_async_copy(
                out_ref.at[pl.ds(t * half, half)], y_stage.at[slot],
                y_sems.at[slot])
            d.start()
            dmas_in[t] = d

        start_in(0)
        start_in(1)

        gmax = state["amax"]
        for off in (1, 2, 3):
            src = lax.rem(my + off, N_DEV)
            r = pltpu.make_async_remote_copy(
                src_ref=amax_tile, dst_ref=amax_gather.at[src],
                send_sem=ax_send_sems.at[0],
                recv_sem=ax_recv_sems.at[src],
                device_id=(left,), device_id_type=pl.DeviceIdType.MESH)
            r.wait_recv()
            gmax = jnp.maximum(gmax, jnp.max(amax_gather[pl.ds(src, 1)]))
        for s in ax_sends:
            s.wait_send()

        scale = gmax / 127.0
        for t in range(n_tiles):
            slot = t % 2
            if 1 <= t and t + 1 < n_tiles:
                dmas_out[t - 1].wait()
                start_in(t + 1)
            dmas_in[t].wait()
            v = y_stage[slot]
            q = jnp.clip(jnp.round(v / scale), -127.0, 127.0) * scale
            y_stage[slot] = q
            d = pltpu.make_async_copy(
                y_stage.at[slot], out_ref.at[pl.ds(t * half, half)],
                y_sems.at[slot])
            d.start()
            dmas_out[t] = d
        dmas_out[n_tiles - 2].wait()
        dmas_out[n_tiles - 1].wait()

    return pl.pallas_call(
        body,
        out_shape=jax.ShapeDtypeStruct((m_glob, n_per), jnp.float32),
        in_specs=[pl.BlockSpec(memory_space=pl.ANY),
                  pl.BlockSpec(memory_space=pl.ANY)],
        out_specs=pl.BlockSpec(memory_space=pl.ANY),
        scratch_shapes=[
            pltpu.VMEM((N_DEV, half, k), jnp.bfloat16),
            pltpu.VMEM((N_DEV, half, k), jnp.bfloat16),
            pltpu.VMEM((k, n_per), jnp.bfloat16),
            pltpu.VMEM((2, half, n_per), jnp.float32),
            pltpu.VMEM((2, xt, k), jnp.float32),
            pltpu.VMEM((2, wt, n_per), jnp.float32),
            pltpu.VMEM((8, 128), jnp.float32),
            pltpu.VMEM((N_DEV, 8, 128), jnp.float32),
            pltpu.SemaphoreType.DMA((6,)),
            pltpu.SemaphoreType.DMA((6,)),
            pltpu.SemaphoreType.DMA((6,)),
            pltpu.SemaphoreType.DMA((6,)),
            pltpu.SemaphoreType.DMA((4,)),
            pltpu.SemaphoreType.DMA((2,)),
            pltpu.SemaphoreType.DMA((3,)),
            pltpu.SemaphoreType.DMA((N_DEV,)),
        ],
        compiler_params=pltpu.CompilerParams(
            collective_id=0,
            vmem_limit_bytes=64 * 1024 * 1024,
        ),
    )(x, w_mat)
